# baseline (speedup 1.0000x reference)
import functools

import jax
import jax.numpy as jnp
from jax import lax
from jax.experimental import pallas as pl
from jax.experimental.pallas import tpu as pltpu
from jax.experimental.pallas import tpu_sc as plsc

_NC = 2
_NS = 16
_NW = _NC * _NS


def kernel(user_inputs, item_inputs, user_table, item_table):
    batch = user_inputs.shape[0]
    embed_dim = user_table.shape[1]
    b_per_w = batch // _NW

    mesh = plsc.VectorSubcoreMesh(core_axis_name="c", subcore_axis_name="s")
    out_type = (
        jax.ShapeDtypeStruct((batch, embed_dim), user_table.dtype),
        jax.ShapeDtypeStruct((batch, embed_dim), item_table.dtype),
    )

    @functools.partial(
        pl.kernel,
        mesh=mesh,
        out_type=out_type,
        scratch_types=[
            pltpu.VMEM((b_per_w,), jnp.int32),
            pltpu.VMEM((b_per_w, embed_dim), jnp.float32),
            pltpu.VMEM((b_per_w,), jnp.int32),
            pltpu.VMEM((b_per_w, embed_dim), jnp.float32),
            pltpu.SemaphoreType.DMA,
            pltpu.SemaphoreType.DMA,
        ],
        compiler_params=pltpu.CompilerParams(use_tc_tiling_on_sc=False),
    )
    def _dual_gather(ut_hbm, it_hbm, ui_hbm, ii_hbm, uo_hbm, io_hbm,
                     uidx_v, urows_v, iidx_v, irows_v, usem, isem):
        wid = lax.axis_index("s") * _NC + lax.axis_index("c")
        base = wid * b_per_w
        pltpu.sync_copy(ui_hbm.at[pl.ds(base, b_per_w)], uidx_v)
        ucopy = pltpu.async_copy(ut_hbm.at[uidx_v], urows_v, usem)
        pltpu.sync_copy(ii_hbm.at[pl.ds(base, b_per_w)], iidx_v)
        icopy = pltpu.async_copy(it_hbm.at[iidx_v], irows_v, isem)
        ucopy.wait()
        pltpu.sync_copy(urows_v, uo_hbm.at[pl.ds(base, b_per_w)])
        icopy.wait()
        pltpu.sync_copy(irows_v, io_hbm.at[pl.ds(base, b_per_w)])

    return _dual_gather(user_table, item_table, user_inputs, item_inputs)


# speedup vs baseline: 1.4964x; 1.4964x over previous
import functools

import jax
import jax.numpy as jnp
from jax import lax
from jax.experimental import pallas as pl
from jax.experimental.pallas import tpu as pltpu
from jax.experimental.pallas import tpu_sc as plsc

_NC = 2
_NS = 16
_NW = _NC * _NS
_CHUNK = 256


def kernel(user_inputs, item_inputs, user_table, item_table):
    batch = user_inputs.shape[0]
    embed_dim = user_table.shape[1]
    b_per_w = batch // _NW
    n_chunks = b_per_w // _CHUNK

    mesh = plsc.VectorSubcoreMesh(core_axis_name="c", subcore_axis_name="s")
    out_type = (
        jax.ShapeDtypeStruct((batch, embed_dim), user_table.dtype),
        jax.ShapeDtypeStruct((batch, embed_dim), item_table.dtype),
    )

    @functools.partial(
        pl.kernel,
        mesh=mesh,
        out_type=out_type,
        scratch_types=[
            pltpu.VMEM((b_per_w,), jnp.int32),
            pltpu.VMEM((b_per_w,), jnp.int32),
            pltpu.VMEM((_CHUNK, embed_dim), jnp.float32),
            pltpu.VMEM((_CHUNK, embed_dim), jnp.float32),
            pltpu.SemaphoreType.DMA,
            pltpu.SemaphoreType.DMA,
        ],
    )
    def _dual_gather(ut_hbm, it_hbm, ui_hbm, ii_hbm, uo_hbm, io_hbm,
                     uidx_v, iidx_v, urows_v, irows_v, usem, isem):
        wid = lax.axis_index("s") * _NC + lax.axis_index("c")
        base = wid * b_per_w

        pltpu.sync_copy(ui_hbm.at[pl.ds(base, b_per_w)], uidx_v)
        pltpu.sync_copy(ii_hbm.at[pl.ds(base, b_per_w)], iidx_v)
        for c in range(n_chunks):
            off = c * _CHUNK

            @pl.loop(0, _CHUNK, step=16)
            def _(j0):
                uvec = uidx_v[pl.ds(off + j0, 16)]
                ivec = iidx_v[pl.ds(off + j0, 16)]
                for l in range(16):
                    pltpu.make_async_copy(
                        ut_hbm.at[pl.ds(uvec[l], 1)],
                        urows_v.at[pl.ds(j0 + l, 1)],
                        usem,
                    ).start()
                    pltpu.make_async_copy(
                        it_hbm.at[pl.ds(ivec[l], 1)],
                        irows_v.at[pl.ds(j0 + l, 1)],
                        isem,
                    ).start()

            pltpu.make_async_copy(
                ut_hbm.at[pl.ds(0, _CHUNK)], urows_v, usem
            ).wait()
            pltpu.sync_copy(urows_v, uo_hbm.at[pl.ds(base + off, _CHUNK)])
            pltpu.make_async_copy(
                it_hbm.at[pl.ds(0, _CHUNK)], irows_v, isem
            ).wait()
            pltpu.sync_copy(irows_v, io_hbm.at[pl.ds(base + off, _CHUNK)])

    return _dual_gather(user_table, item_table, user_inputs, item_inputs)


# speedup vs baseline: 1.5003x; 1.0026x over previous
import functools

import jax
import jax.numpy as jnp
from jax import lax
from jax.experimental import pallas as pl
from jax.experimental.pallas import tpu as pltpu
from jax.experimental.pallas import tpu_sc as plsc

_NC = 2
_NS = 16
_NW = _NC * _NS
_CHUNK = 256


def kernel(user_inputs, item_inputs, user_table, item_table):
    batch = user_inputs.shape[0]
    embed_dim = user_table.shape[1]
    b_per_w = batch // _NW
    n_chunks = b_per_w // _CHUNK

    mesh = plsc.VectorSubcoreMesh(core_axis_name="c", subcore_axis_name="s")
    out_type = (
        jax.ShapeDtypeStruct((batch, embed_dim), user_table.dtype),
        jax.ShapeDtypeStruct((batch, embed_dim), item_table.dtype),
    )

    @functools.partial(
        pl.kernel,
        mesh=mesh,
        out_type=out_type,
        scratch_types=[
            pltpu.VMEM((b_per_w,), jnp.int32),
            pltpu.VMEM((b_per_w,), jnp.int32),
            pltpu.VMEM((_CHUNK, embed_dim), jnp.float32),
            pltpu.VMEM((_CHUNK, embed_dim), jnp.float32),
            pltpu.SemaphoreType.DMA,
            pltpu.SemaphoreType.DMA,
        ],
    )
    def _dual_gather(ut_hbm, it_hbm, ui_hbm, ii_hbm, uo_hbm, io_hbm,
                     uidx_v, iidx_v, urows_v, irows_v, usem, isem):
        wid = lax.axis_index("s") * _NC + lax.axis_index("c")
        base = wid * b_per_w

        pltpu.sync_copy(ui_hbm.at[pl.ds(base, b_per_w)], uidx_v)
        pltpu.sync_copy(ii_hbm.at[pl.ds(base, b_per_w)], iidx_v)
        for c in range(n_chunks):
            off = c * _CHUNK

            @plsc.parallel_loop(0, _CHUNK, step=16, unroll=2)
            def _(j0):
                uvec = uidx_v[pl.ds(off + j0, 16)]
                ivec = iidx_v[pl.ds(off + j0, 16)]
                for l in range(16):
                    pltpu.make_async_copy(
                        ut_hbm.at[pl.ds(uvec[l], 1)],
                        urows_v.at[pl.ds(j0 + l, 1)],
                        usem,
                    ).start()
                    pltpu.make_async_copy(
                        it_hbm.at[pl.ds(ivec[l], 1)],
                        irows_v.at[pl.ds(j0 + l, 1)],
                        isem,
                    ).start()

            pltpu.make_async_copy(
                ut_hbm.at[pl.ds(0, _CHUNK)], urows_v, usem
            ).wait()
            pltpu.sync_copy(urows_v, uo_hbm.at[pl.ds(base + off, _CHUNK)])
            pltpu.make_async_copy(
                it_hbm.at[pl.ds(0, _CHUNK)], irows_v, isem
            ).wait()
            pltpu.sync_copy(irows_v, io_hbm.at[pl.ds(base + off, _CHUNK)])

    return _dual_gather(user_table, item_table, user_inputs, item_inputs)
